# baseline (speedup 1.0000x reference)
import functools

import jax
import jax.numpy as jnp
from jax import lax
from jax.experimental import pallas as pl
from jax.experimental import pallas as _pl
from jax.experimental.pallas import tpu as pltpu
from jax.experimental.pallas import tpu_sc as plsc

T = 2048
H = 1024
FF = 1024
E = 8
BT = 256
NT = T // BT + E - 1
PAD_T = NT * BT
NW = 32


def _router_body(x_ref, wr_ref, xs_ref, eidx_ref):
    x = x_ref[...]
    logits = lax.dot_general(
        x, wr_ref[...], (((1,), (1,)), ((), ())),
        preferred_element_type=jnp.float32,
    )
    topv = jnp.max(logits, axis=1, keepdims=True)
    ids = lax.broadcasted_iota(jnp.int32, logits.shape, 1)
    eidx_ref[...] = jnp.min(jnp.where(logits == topv, ids, E), axis=1,
                            keepdims=True)
    xs_ref[...] = x * jax.nn.sigmoid(topv)


def _swiglu(xb, wg, wu, wd):
    nt = (((1,), (1,)), ((), ()))
    g = lax.dot_general(xb, wg, nt, preferred_element_type=jnp.float32)
    u = lax.dot_general(xb, wu, nt, preferred_element_type=jnp.float32)
    act = (g * jax.nn.sigmoid(g) * u).astype(jnp.bfloat16)
    return lax.dot_general(act, wd, nt, preferred_element_type=jnp.float32)


def _moe_ffn_body(te_ref, xp_ref, x_ref, wg_ref, wu_ref, wd_ref,
                  wsg_ref, wsu_ref, wsd_ref, yr_ref, ysh_ref):
    del te_ref
    i = pl.program_id(0)

    @pl.when(i < NT)
    def _():
        yr_ref[...] = _swiglu(xp_ref[...].astype(jnp.bfloat16),
                              wg_ref[0].astype(jnp.bfloat16),
                              wu_ref[0].astype(jnp.bfloat16),
                              wd_ref[0].astype(jnp.bfloat16))

    @pl.when(i >= NT)
    def _():
        ysh_ref[...] = _swiglu(x_ref[...].astype(jnp.bfloat16),
                               wsg_ref[...].astype(jnp.bfloat16),
                               wsu_ref[...].astype(jnp.bfloat16),
                               wsd_ref[...].astype(jnp.bfloat16))


def _add_body(a_ref, b_ref, o_ref):
    o_ref[...] = a_ref[...] + b_ref[...]


def _sc_worker_id():
    return lax.axis_index("s") * 2 + lax.axis_index("c")


def _make_row_gather(table_rows, out_rows, n_workers, chunk):
    rows_w = out_rows // n_workers
    nch = rows_w // chunk
    assert nch <= 8
    mesh = plsc.VectorSubcoreMesh(core_axis_name="c", subcore_axis_name="s")

    @functools.partial(
        pl.kernel, mesh=mesh,
        out_type=jax.ShapeDtypeStruct((out_rows, H), jnp.float32),
        scratch_types=[
            pltpu.VMEM((8, chunk), jnp.int32),
            [pltpu.VMEM((chunk, H), jnp.float32) for _ in range(2)],
            [pltpu.SemaphoreType.DMA for _ in range(2)],
            [pltpu.SemaphoreType.DMA for _ in range(2)],
        ],
    )
    def gather_k(table_hbm, idx_hbm, out_hbm, idx_v, bufs, gsems, ssems):
        wid = _sc_worker_id()

        @pl.when(wid < n_workers)
        def _():
            off = wid * rows_w
            pltpu.sync_copy(idx_hbm.at[pl.ds(wid * 8, 8)], idx_v)
            gets = [None, None]
            puts = [None, None]
            for c in range(nch):
                b = c % 2
                if puts[b] is not None:
                    puts[b].wait()
                gets[b] = pltpu.async_copy(
                    table_hbm.at[idx_v.at[c]], bufs[b], gsems[b])
                if c > 0:
                    pb = (c - 1) % 2
                    gets[pb].wait()
                    puts[pb] = pltpu.async_copy(
                        bufs[pb],
                        out_hbm.at[pl.ds(off + (c - 1) * chunk, chunk)],
                        ssems[pb])
            lb = (nch - 1) % 2
            gets[lb].wait()
            pltpu.async_copy(bufs[lb],
                             out_hbm.at[pl.ds(off + (nch - 1) * chunk, chunk)],
                             ssems[lb]).wait()
            if puts[1 - lb] is not None:
                puts[1 - lb].wait()

    return gather_k


def _make_row_scatter(out_rows, n_workers, chunk):
    src_rows_w = None
    mesh = plsc.VectorSubcoreMesh(core_axis_name="c", subcore_axis_name="s")

    @functools.partial(
        pl.kernel, mesh=mesh,
        out_type=jax.ShapeDtypeStruct((out_rows, H), jnp.float32),
        scratch_types=[
            pltpu.VMEM((8, chunk), jnp.int32),
            [pltpu.VMEM((chunk, H), jnp.float32) for _ in range(2)],
            [pltpu.SemaphoreType.DMA for _ in range(2)],
            [pltpu.SemaphoreType.DMA for _ in range(2)],
        ],
    )
    def scatter_k(src_hbm, idx_hbm, out_hbm, idx_v, bufs, gsems, ssems):
        rows_w = src_hbm.shape[0] // n_workers
        nch = rows_w // chunk
        wid = _sc_worker_id()

        @pl.when(wid < n_workers)
        def _():
            off = wid * rows_w
            pltpu.sync_copy(idx_hbm.at[pl.ds(wid * 8, 8)], idx_v)
            gets = [None, None]
            puts = [None, None]
            for c in range(nch):
                b = c % 2
                if puts[b] is not None:
                    puts[b].wait()
                gets[b] = pltpu.async_copy(
                    src_hbm.at[pl.ds(off + c * chunk, chunk)], bufs[b],
                    gsems[b])
                if c > 0:
                    pb = (c - 1) % 2
                    gets[pb].wait()
                    puts[pb] = pltpu.async_copy(
                        bufs[pb], out_hbm.at[idx_v.at[c - 1]], ssems[pb])
            lb = (nch - 1) % 2
            gets[lb].wait()
            pltpu.async_copy(bufs[lb], out_hbm.at[idx_v.at[nch - 1]],
                             ssems[lb]).wait()
            if puts[1 - lb] is not None:
                puts[1 - lb].wait()

    return scatter_k


def _pad_idx_rows(idx_flat, n_workers, chunk):
    nch = idx_flat.shape[0] // (n_workers * chunk)
    idx3 = idx_flat.reshape(n_workers, nch, chunk)
    return jnp.pad(idx3, ((0, 0), (0, 8 - nch), (0, 0))).reshape(
        n_workers * 8, chunk)


def kernel(hidden_states, Wr, Wg, Wu, Wd, Wsg, Wsu, Wsd):
    x = hidden_states

    xs, eidx = pl.pallas_call(
        _router_body,
        out_shape=[
            jax.ShapeDtypeStruct((T, H), jnp.float32),
            jax.ShapeDtypeStruct((T, 1), jnp.int32),
        ],
    )(x, Wr)

    e = eidx[:, 0]
    oh = (e[:, None] == jnp.arange(E, dtype=jnp.int32)[None, :]).astype(
        jnp.int32)
    csum = jnp.cumsum(oh, axis=0)
    counts = csum[-1]
    rank = jnp.take_along_axis(csum, e[:, None], axis=1)[:, 0] - 1
    pcnt = ((counts + BT - 1) // BT) * BT
    pcum = jnp.cumsum(pcnt).astype(jnp.int32)
    pstart = pcum - pcnt
    pos = pstart[e] + rank
    pos_rows = _pad_idx_rows(pos, NW, 16)
    te = jnp.clip(
        jnp.searchsorted(pcum, jnp.arange(NT, dtype=jnp.int32) * BT,
                         side="right"),
        0, E - 1).astype(jnp.int32)

    xp = _make_row_scatter(PAD_T, NW, 16)(xs, pos_rows)

    te_full = jnp.concatenate(
        [te, jnp.broadcast_to(te[-1:], (T // BT,))])
    grid_spec = pltpu.PrefetchScalarGridSpec(
        num_scalar_prefetch=1,
        grid=(NT + T // BT,),
        in_specs=[
            pl.BlockSpec((BT, H), lambda i, te_r: (jnp.minimum(i, NT - 1), 0)),
            pl.BlockSpec((BT, H), lambda i, te_r: (jnp.maximum(i - NT, 0), 0)),
            pl.BlockSpec((1, FF, H), lambda i, te_r: (te_r[i], 0, 0)),
            pl.BlockSpec((1, FF, H), lambda i, te_r: (te_r[i], 0, 0)),
            pl.BlockSpec((1, H, FF), lambda i, te_r: (te_r[i], 0, 0)),
            pl.BlockSpec((FF, H), lambda i, te_r: (0, 0)),
            pl.BlockSpec((FF, H), lambda i, te_r: (0, 0)),
            pl.BlockSpec((H, FF), lambda i, te_r: (0, 0)),
        ],
        out_specs=[
            pl.BlockSpec((BT, H), lambda i, te_r: (jnp.minimum(i, NT - 1), 0)),
            pl.BlockSpec((BT, H), lambda i, te_r: (jnp.maximum(i - NT, 0), 0)),
        ],
    )
    yr, ysh = pl.pallas_call(
        _moe_ffn_body,
        grid_spec=grid_spec,
        out_shape=[
            jax.ShapeDtypeStruct((PAD_T, H), jnp.float32),
            jax.ShapeDtypeStruct((T, H), jnp.float32),
        ],
    )(te_full, xp, x, Wg, Wu, Wd, Wsg, Wsu, Wsd)

    ygat = _make_row_gather(PAD_T, T, NW, 16)(yr, pos_rows)

    return pl.pallas_call(
        _add_body,
        grid=(T // BT,),
        in_specs=[
            pl.BlockSpec((BT, H), lambda i: (i, 0)),
            pl.BlockSpec((BT, H), lambda i: (i, 0)),
        ],
        out_specs=pl.BlockSpec((BT, H), lambda i: (i, 0)),
        out_shape=jax.ShapeDtypeStruct((T, H), jnp.float32),
    )(ysh, ygat)


# speedup vs baseline: 1.1461x; 1.1461x over previous
import functools

import jax
import jax.numpy as jnp
from jax import lax
from jax.experimental import pallas as pl
from jax.experimental import pallas as _pl
from jax.experimental.pallas import tpu as pltpu
from jax.experimental.pallas import tpu_sc as plsc

T = 2048
H = 1024
FF = 1024
E = 8
BT = 256
NT = T // BT + E - 1
PAD_T = NT * BT
NW = 32


def _router_body(x_ref, wr_ref, xs_ref, eidx_ref):
    x = x_ref[...]
    logits = lax.dot_general(
        x, wr_ref[...], (((1,), (1,)), ((), ())),
        preferred_element_type=jnp.float32,
    )
    topv = jnp.max(logits, axis=1, keepdims=True)
    ids = lax.broadcasted_iota(jnp.int32, logits.shape, 1)
    eidx_ref[...] = jnp.min(jnp.where(logits == topv, ids, E), axis=1,
                            keepdims=True)
    xs_ref[...] = x * jax.nn.sigmoid(topv)


def _swiglu(xb, wg, wu, wd):
    nt = (((1,), (1,)), ((), ()))
    g = lax.dot_general(xb, wg, nt, preferred_element_type=jnp.float32)
    u = lax.dot_general(xb, wu, nt, preferred_element_type=jnp.float32)
    act = (g * jax.nn.sigmoid(g) * u).astype(jnp.bfloat16)
    return lax.dot_general(act, wd, nt, preferred_element_type=jnp.float32)


def _routed_ffn_body(te_ref, xp_ref, wg_ref, wu_ref, wd_ref, out_ref):
    del te_ref
    out_ref[...] = _swiglu(xp_ref[...].astype(jnp.bfloat16),
                           wg_ref[0].astype(jnp.bfloat16),
                           wu_ref[0].astype(jnp.bfloat16),
                           wd_ref[0].astype(jnp.bfloat16))


def _shared_ffn_body(x_ref, wg_ref, wu_ref, wd_ref, out_ref):
    xb = x_ref[...].astype(jnp.bfloat16)
    out_ref[...] = _swiglu(xb, wg_ref[...].astype(jnp.bfloat16),
                           wu_ref[...].astype(jnp.bfloat16),
                           wd_ref[...].astype(jnp.bfloat16))


def _add_body(a_ref, b_ref, o_ref):
    o_ref[...] = a_ref[...] + b_ref[...]


def _sc_worker_id():
    return lax.axis_index("s") * 2 + lax.axis_index("c")


def _make_row_gather(table_rows, out_rows, n_workers, chunk):
    rows_w = out_rows // n_workers
    nch = rows_w // chunk
    assert nch <= 8
    mesh = plsc.VectorSubcoreMesh(core_axis_name="c", subcore_axis_name="s")

    @functools.partial(
        pl.kernel, mesh=mesh,
        out_type=jax.ShapeDtypeStruct((out_rows, H), jnp.float32),
        scratch_types=[
            pltpu.VMEM((8, chunk), jnp.int32),
            [pltpu.VMEM((chunk, H), jnp.float32) for _ in range(2)],
            [pltpu.SemaphoreType.DMA for _ in range(2)],
            [pltpu.SemaphoreType.DMA for _ in range(2)],
        ],
    )
    def gather_k(table_hbm, idx_hbm, out_hbm, idx_v, bufs, gsems, ssems):
        wid = _sc_worker_id()

        @pl.when(wid < n_workers)
        def _():
            off = wid * rows_w
            pltpu.sync_copy(idx_hbm.at[pl.ds(wid * 8, 8)], idx_v)
            gets = [None, None]
            puts = [None, None]
            for c in range(nch):
                b = c % 2
                if puts[b] is not None:
                    puts[b].wait()
                gets[b] = pltpu.async_copy(
                    table_hbm.at[idx_v.at[c]], bufs[b], gsems[b])
                if c > 0:
                    pb = (c - 1) % 2
                    gets[pb].wait()
                    puts[pb] = pltpu.async_copy(
                        bufs[pb],
                        out_hbm.at[pl.ds(off + (c - 1) * chunk, chunk)],
                        ssems[pb])
            lb = (nch - 1) % 2
            gets[lb].wait()
            pltpu.async_copy(bufs[lb],
                             out_hbm.at[pl.ds(off + (nch - 1) * chunk, chunk)],
                             ssems[lb]).wait()
            if puts[1 - lb] is not None:
                puts[1 - lb].wait()

    return gather_k


def _make_row_scatter(out_rows, n_workers, chunk):
    src_rows_w = None
    mesh = plsc.VectorSubcoreMesh(core_axis_name="c", subcore_axis_name="s")

    @functools.partial(
        pl.kernel, mesh=mesh,
        out_type=jax.ShapeDtypeStruct((out_rows, H), jnp.float32),
        scratch_types=[
            pltpu.VMEM((8, chunk), jnp.int32),
            [pltpu.VMEM((chunk, H), jnp.float32) for _ in range(2)],
            [pltpu.SemaphoreType.DMA for _ in range(2)],
            [pltpu.SemaphoreType.DMA for _ in range(2)],
        ],
    )
    def scatter_k(src_hbm, idx_hbm, out_hbm, idx_v, bufs, gsems, ssems):
        rows_w = src_hbm.shape[0] // n_workers
        nch = rows_w // chunk
        wid = _sc_worker_id()

        @pl.when(wid < n_workers)
        def _():
            off = wid * rows_w
            pltpu.sync_copy(idx_hbm.at[pl.ds(wid * 8, 8)], idx_v)
            gets = [None, None]
            puts = [None, None]
            for c in range(nch):
                b = c % 2
                if puts[b] is not None:
                    puts[b].wait()
                gets[b] = pltpu.async_copy(
                    src_hbm.at[pl.ds(off + c * chunk, chunk)], bufs[b],
                    gsems[b])
                if c > 0:
                    pb = (c - 1) % 2
                    gets[pb].wait()
                    puts[pb] = pltpu.async_copy(
                        bufs[pb], out_hbm.at[idx_v.at[c - 1]], ssems[pb])
            lb = (nch - 1) % 2
            gets[lb].wait()
            pltpu.async_copy(bufs[lb], out_hbm.at[idx_v.at[nch - 1]],
                             ssems[lb]).wait()
            if puts[1 - lb] is not None:
                puts[1 - lb].wait()

    return scatter_k


def _pad_idx_rows(idx_flat, n_workers, chunk):
    nch = idx_flat.shape[0] // (n_workers * chunk)
    idx3 = idx_flat.reshape(n_workers, nch, chunk)
    return jnp.pad(idx3, ((0, 0), (0, 8 - nch), (0, 0))).reshape(
        n_workers * 8, chunk)


def kernel(hidden_states, Wr, Wg, Wu, Wd, Wsg, Wsu, Wsd):
    x = hidden_states

    xs, eidx = pl.pallas_call(
        _router_body,
        out_shape=[
            jax.ShapeDtypeStruct((T, H), jnp.float32),
            jax.ShapeDtypeStruct((T, 1), jnp.int32),
        ],
    )(x, Wr)

    e = eidx[:, 0]
    oh = (e[:, None] == jnp.arange(E, dtype=jnp.int32)[None, :]).astype(
        jnp.int32)
    csum = jnp.cumsum(oh, axis=0)
    counts = csum[-1]
    pcnt = ((counts + BT - 1) // BT) * BT
    pcum = jnp.cumsum(pcnt).astype(jnp.int32)
    pstart = pcum - pcnt
    pos = jnp.sum(oh * (csum + pstart[None, :]), axis=1) - 1
    pos_rows = _pad_idx_rows(pos, NW, 16)
    tile_lo = jnp.arange(NT, dtype=jnp.int32) * BT
    te = jnp.minimum(
        jnp.sum((pcum[None, :] <= tile_lo[:, None]).astype(jnp.int32),
                axis=1),
        E - 1).astype(jnp.int32)

    xp = _make_row_scatter(PAD_T, NW, 16)(xs, pos_rows)

    ysh = pl.pallas_call(
        _shared_ffn_body,
        grid=(T // BT,),
        in_specs=[
            pl.BlockSpec((BT, H), lambda i: (i, 0)),
            pl.BlockSpec((FF, H), lambda i: (0, 0)),
            pl.BlockSpec((FF, H), lambda i: (0, 0)),
            pl.BlockSpec((H, FF), lambda i: (0, 0)),
        ],
        out_specs=pl.BlockSpec((BT, H), lambda i: (i, 0)),
        out_shape=jax.ShapeDtypeStruct((T, H), jnp.float32),
    )(x, Wsg, Wsu, Wsd)

    grid_spec = pltpu.PrefetchScalarGridSpec(
        num_scalar_prefetch=1,
        grid=(NT,),
        in_specs=[
            pl.BlockSpec((BT, H), lambda i, te_r: (i, 0)),
            pl.BlockSpec((1, FF, H), lambda i, te_r: (te_r[i], 0, 0)),
            pl.BlockSpec((1, FF, H), lambda i, te_r: (te_r[i], 0, 0)),
            pl.BlockSpec((1, H, FF), lambda i, te_r: (te_r[i], 0, 0)),
        ],
        out_specs=pl.BlockSpec((BT, H), lambda i, te_r: (i, 0)),
    )
    yr = pl.pallas_call(
        _routed_ffn_body,
        grid_spec=grid_spec,
        out_shape=jax.ShapeDtypeStruct((PAD_T, H), jnp.float32),
    )(te, xp, Wg, Wu, Wd)

    ygat = _make_row_gather(PAD_T, T, NW, 16)(yr, pos_rows)

    return pl.pallas_call(
        _add_body,
        grid=(T // BT,),
        in_specs=[
            pl.BlockSpec((BT, H), lambda i: (i, 0)),
            pl.BlockSpec((BT, H), lambda i: (i, 0)),
        ],
        out_specs=pl.BlockSpec((BT, H), lambda i: (i, 0)),
        out_shape=jax.ShapeDtypeStruct((T, H), jnp.float32),
    )(ysh, ygat)
